# baseline (speedup 1.0000x reference)
import functools

import jax
import jax.numpy as jnp
from jax import lax
from jax.experimental import pallas as pl
from jax.experimental.pallas import tpu as pltpu
from jax.experimental.pallas import tpu_sc as plsc

N = 10000
E = 160000
D = 128
NP = 10240
EP = 163840
EB = EP // 128
NC = 2
NS = 16
TPB = EB // (NC * NS)
NPT = NP // NS
PB = EB // NS


BN = 1024
GRID = NP // BN


def _tc_pre_body(rnn_ref, emb_ref, w0_ref, b0_ref, qt_ref, wet_ref, db_ref,
                 wyr_ref, byr_ref, x1_ref, pz_ref, yr_ref):
    r = rnn_ref[...]
    x1_ref[...] = jnp.dot(r, w0_ref[...], preferred_element_type=jnp.float32) + b0_ref[...]
    db = db_ref[...]
    pz_ref[pl.ds(0, 2), :] = lax.dot_general(
        qt_ref[...], r, (((1,), (1,)), ((), ())),
        preferred_element_type=jnp.float32) + db[0:2]
    pz_ref[pl.ds(2, 2), :] = lax.dot_general(
        wet_ref[...], emb_ref[...], (((1,), (1,)), ((), ())),
        preferred_element_type=jnp.float32) + db[2:4]
    yr_ref[...] = jnp.dot(r, wyr_ref[...], preferred_element_type=jnp.float32) + byr_ref[...]


def _tc_pre(rnn, emb, w0t, b0, qt2, wet2, db, wyr, byr):
    return pl.pallas_call(
        _tc_pre_body,
        grid=(GRID,),
        in_specs=[
            pl.BlockSpec((BN, D), lambda i: (i, 0)),
            pl.BlockSpec((BN, 16), lambda i: (i, 0)),
            pl.BlockSpec((D, D), lambda i: (0, 0)),
            pl.BlockSpec((1, D), lambda i: (0, 0)),
            pl.BlockSpec((2, D), lambda i: (0, 0)),
            pl.BlockSpec((2, 16), lambda i: (0, 0)),
            pl.BlockSpec((4, 1), lambda i: (0, 0)),
            pl.BlockSpec((D, 8), lambda i: (0, 0)),
            pl.BlockSpec((1, 8), lambda i: (0, 0)),
        ],
        out_specs=[
            pl.BlockSpec((BN, D), lambda i: (i, 0)),
            pl.BlockSpec((4, BN), lambda i: (0, i)),
            pl.BlockSpec((BN, 8), lambda i: (i, 0)),
        ],
        out_shape=[
            jax.ShapeDtypeStruct((NP, D), jnp.float32),
            jax.ShapeDtypeStruct((4, NP), jnp.float32),
            jax.ShapeDtypeStruct((NP, 8), jnp.float32),
        ],
    )(rnn, emb, w0t, b0, qt2, wet2, db, wyr, byr)


def _tc_mid_body(x1_ref, a0_ref, a1_ref, w_ref, b_ref, x2_ref):
    g = a0_ref[0] + a1_ref[0] + x1_ref[...]
    x2_ref[...] = jnp.dot(g, w_ref[...], preferred_element_type=jnp.float32) + b_ref[...]


def _tc_mid(x1, parts, w1t, b1):
    return pl.pallas_call(
        _tc_mid_body,
        grid=(GRID,),
        in_specs=[
            pl.BlockSpec((BN, D), lambda i: (i, 0)),
            pl.BlockSpec((1, BN, D), lambda i: (0, i, 0)),
            pl.BlockSpec((1, BN, D), lambda i: (1, i, 0)),
            pl.BlockSpec((D, D), lambda i: (0, 0)),
            pl.BlockSpec((1, D), lambda i: (0, 0)),
        ],
        out_specs=pl.BlockSpec((BN, D), lambda i: (i, 0)),
        out_shape=jax.ShapeDtypeStruct((NP, D), jnp.float32),
    )(x1, parts, parts, w1t, b1)


def _tc_fin_body(x2_ref, b0_ref, b1_ref, yr_ref, wf_ref, bf_ref, y_ref):
    g = b0_ref[0] + b1_ref[0] + x2_ref[...]
    y_ref[...] = (yr_ref[...] + bf_ref[...]
                  + jnp.dot(g, wf_ref[...], preferred_element_type=jnp.float32))


def _tc_fin(x2, parts, yr8, wf8, bf8):
    return pl.pallas_call(
        _tc_fin_body,
        grid=(GRID,),
        in_specs=[
            pl.BlockSpec((BN, D), lambda i: (i, 0)),
            pl.BlockSpec((1, BN, D), lambda i: (0, i, 0)),
            pl.BlockSpec((1, BN, D), lambda i: (1, i, 0)),
            pl.BlockSpec((BN, 8), lambda i: (i, 0)),
            pl.BlockSpec((D, 8), lambda i: (0, 0)),
            pl.BlockSpec((1, 8), lambda i: (0, 0)),
        ],
        out_specs=pl.BlockSpec((BN, 8), lambda i: (i, 0)),
        out_shape=jax.ShapeDtypeStruct((NP, 8), jnp.float32),
    )(x2, parts, parts, yr8, wf8, bf8)


_sc_mesh = plsc.VectorSubcoreMesh(core_axis_name="c", subcore_axis_name="s")


def _gate_body(pz, cent, consts, srcb, dstb, ewb, gate_out,
               ps_v, pd_v, zs_v, zd_v, cent_v, src_all, dst_all, ew_all,
               tmp, au_s, au_d, zb, cv, src2, dst2, grow, riota, us_sh, ud_sh):
    cid = lax.axis_index("c")
    sid = lax.axis_index("s")
    pltpu.sync_copy(pz.at[0], ps_v)
    pltpu.sync_copy(pz.at[1], pd_v)
    pltpu.sync_copy(pz.at[2], zs_v)
    pltpu.sync_copy(pz.at[3], zd_v)
    pltpu.sync_copy(cent, cent_v)
    pltpu.sync_copy(consts, cv)
    pb0 = sid * PB
    pltpu.sync_copy(srcb.at[pl.ds(pb0, PB)], src_all)
    pltpu.sync_copy(dstb.at[pl.ds(pb0, PB)], dst_all)
    pltpu.sync_copy(ewb.at[pl.ds(pb0, PB)], ew_all)
    zero16 = jnp.zeros((16,), jnp.float32)
    lane = lax.iota(jnp.int32, 16)
    for r in range(NPT // 128):
        for m in range(8):
            zb[r, pl.ds(m * 16, 16)] = zero16
    for r in range(5):
        riota[pl.ds(r * 16, 16)] = lane + (r * 16)
    shsl = pl.ds(sid * (NPT // 128), NPT // 128)
    pltpu.sync_copy(zb, us_sh.at[shsl])
    pltpu.sync_copy(zb, ud_sh.at[shsl])

    def zero_au():
        def zrow(r, carry):
            for m in range(8):
                sl = pl.ds(m * 16, 16)
                au_s[r, sl] = zero16
                au_d[r, sl] = zero16
            return carry

        lax.fori_loop(0, NP // 128, zrow, 0)

    zero_au()
    plsc.subcore_barrier()

    def edge_pass(j, carry):
        for m in range(8):
            sl = pl.ds(m * 16, 16)
            s16 = src_all[j, sl]
            e16 = ew_all[j, sl]
            d16 = dst_all[j, sl]
            drow = lax.shift_right_logical(d16, 7)
            dcol = lax.bitwise_and(d16, 127)
            plsc.addupdate_scatter(au_s, [drow, dcol],
                                   plsc.load_gather(ps_v, [s16]) * e16)
            plsc.addupdate_scatter(au_d, [drow, dcol],
                                   plsc.load_gather(pd_v, [s16]) * e16)
        return carry

    def reduce_au():
        pltpu.sync_copy(au_s, us_sh.at[riota], add=True)
        pltpu.sync_copy(au_d, ud_sh.at[riota], add=True)

    lax.fori_loop(0, PB, edge_pass, 0)
    reduce_au()
    plsc.subcore_barrier()

    cvec = cv[pl.ds(0, 16)]
    cs = cvec[0]
    cd = cvec[1]
    pltpu.sync_copy(us_sh, tmp)

    def add_c(r, carry):
        for m in range(8):
            sl = pl.ds(m * 16, 16)
            fl = pl.ds(r * 128 + m * 16, 16)
            ps_v[fl] = ps_v[fl] + tmp[r, sl] + cs
        return carry

    lax.fori_loop(0, NP // 128, add_c, 0)
    pltpu.sync_copy(ud_sh, tmp)

    def add_cd(r, carry):
        for m in range(8):
            sl = pl.ds(m * 16, 16)
            fl = pl.ds(r * 128 + m * 16, 16)
            pd_v[fl] = pd_v[fl] + tmp[r, sl] + cd
        return carry

    lax.fori_loop(0, NP // 128, add_cd, 0)
    plsc.subcore_barrier()
    pltpu.sync_copy(zb, us_sh.at[shsl])
    pltpu.sync_copy(zb, ud_sh.at[shsl])
    zero_au()
    plsc.subcore_barrier()

    lax.fori_loop(0, PB, edge_pass, 0)
    reduce_au()
    plsc.subcore_barrier()

    pltpu.sync_copy(us_sh, tmp)

    def add_z(r, carry):
        for m in range(8):
            sl = pl.ds(m * 16, 16)
            fl = pl.ds(r * 128 + m * 16, 16)
            c16 = cent_v[fl]
            ps_v[fl] = ps_v[fl] + tmp[r, sl] + plsc.load_gather(zs_v, [c16])
        return carry

    lax.fori_loop(0, NP // 128, add_z, 0)
    pltpu.sync_copy(ud_sh, tmp)

    def add_zd(r, carry):
        for m in range(8):
            sl = pl.ds(m * 16, 16)
            fl = pl.ds(r * 128 + m * 16, 16)
            c16 = cent_v[fl]
            pd_v[fl] = pd_v[fl] + tmp[r, sl] + plsc.load_gather(zd_v, [c16])
        return carry

    lax.fori_loop(0, NP // 128, add_zd, 0)

    gb0 = cid * (EB // NC) + sid * TPB
    pltpu.sync_copy(srcb.at[pl.ds(gb0, TPB)], src2)
    pltpu.sync_copy(dstb.at[pl.ds(gb0, TPB)], dst2)
    lane = lax.iota(jnp.int32, 16)

    def gate_blk(j, carry):
        base = (gb0 + j) * 128
        for m in range(8):
            sl = pl.ds(m * 16, 16)
            s16 = plsc.load_gather(ps_v, [src2[j, sl]])
            t16 = plsc.load_gather(pd_v, [dst2[j, sl]])
            g = jnp.maximum(s16 + t16, 0.0)
            eidx = lane + (base + m * 16)
            grow[sl] = jnp.where(eidx < E, g, 0.0)
        pltpu.sync_copy(grow, gate_out.at[gb0 + j])
        return carry

    lax.fori_loop(0, TPB, gate_blk, 0)


_gate_kernel = functools.partial(
    pl.kernel,
    _gate_body,
    out_type=jax.ShapeDtypeStruct((EB, 128), jnp.float32),
    mesh=_sc_mesh,
    compiler_params=pltpu.CompilerParams(needs_layout_passes=False),
    scratch_types=[
        pltpu.VMEM((NP,), jnp.float32),
        pltpu.VMEM((NP,), jnp.float32),
        pltpu.VMEM((NP,), jnp.float32),
        pltpu.VMEM((NP,), jnp.float32),
        pltpu.VMEM((NP,), jnp.int32),
        pltpu.VMEM((PB, 128), jnp.int32),
        pltpu.VMEM((PB, 128), jnp.int32),
        pltpu.VMEM((PB, 128), jnp.float32),
        pltpu.VMEM((NP // 128, 128), jnp.float32),
        pltpu.VMEM((NP // 128, 128), jnp.float32),
        pltpu.VMEM((NP // 128, 128), jnp.float32),
        pltpu.VMEM((NPT // 128, 128), jnp.float32),
        pltpu.VMEM((16,), jnp.float32),
        pltpu.VMEM((TPB, 128), jnp.int32),
        pltpu.VMEM((TPB, 128), jnp.int32),
        pltpu.VMEM((128,), jnp.float32),
        pltpu.VMEM((80,), jnp.int32),
        pltpu.VMEM_SHARED((NP // 128, 128), jnp.float32),
        pltpu.VMEM_SHARED((NP // 128, 128), jnp.float32),
    ],
)()


CB = 64
CPT = EP // (NC * NS * CB)


def _conv_body(x_hbm, srcb, dstb, gateb, out_hbm,
               src_v, dst_v, gate_v, rows0, rows1, rows2, acc,
               g0, g1, g2, s0, s1, s2):
    cid = lax.axis_index("c")
    sid = lax.axis_index("s")
    wid = cid * NS + sid
    nsl = pl.ds(sid * NPT, NPT)
    zero16 = jnp.zeros((16,), jnp.float32)
    for k in range(CB):
        for m in range(8):
            rows0[k, pl.ds(m * 16, 16)] = zero16
    for i in range(NPT // CB):
        pltpu.sync_copy(rows0, acc.at[pl.ds(sid * NPT + i * CB, CB)])
    pltpu.sync_copy(srcb.at[pl.ds(wid * TPB, TPB)], src_v)
    pltpu.sync_copy(dstb.at[pl.ds(wid * CPT, CPT)], dst_v)
    pltpu.sync_copy(gateb.at[pl.ds(wid * TPB, TPB)], gate_v)
    plsc.subcore_barrier()

    rows = [rows0, rows1, rows2]
    gsem = [g0, g1, g2]
    ssem = [s0, s1, s2]

    def scale(buf, j):
        def row16(c, c2):
            gvec = gate_v[j // 2, pl.ds((j % 2) * CB + c * 16, 16)]
            for l in range(16):
                g = gvec[l]
                k = c * 16 + l
                for m in range(8):
                    sl = pl.ds(m * 16, 16)
                    buf[k, sl] = buf[k, sl] * g
            return c2

        lax.fori_loop(0, CB // 16, row16, 0)

    def slot(j, b, static=False):
        bn = (b + 1) % 3
        pltpu.make_async_copy(x_hbm.at[src_v.at[j // 2, pl.ds((j % 2) * CB, CB)]], rows[b], gsem[b]).wait()
        scale(rows[b], j)
        pltpu.async_copy(rows[b], acc.at[dst_v.at[j]], ssem[b], add=True)

        def drain():
            pltpu.make_async_copy(rows[bn], acc.at[dst_v.at[j]], ssem[bn]).wait()

        def fetch():
            pltpu.async_copy(x_hbm.at[src_v.at[(j + 1) // 2, pl.ds(((j + 1) % 2) * CB, CB)]], rows[bn], gsem[bn])

        if static:
            if j >= 2:
                drain()
            if j >= 1 and j + 1 < CPT:
                fetch()
        else:
            pl.when(j >= 2)(drain)
            pl.when(jnp.logical_and(j >= 1, j + 1 < CPT))(fetch)

    pltpu.async_copy(x_hbm.at[src_v.at[0, pl.ds(0, CB)]], rows0, g0)
    pltpu.async_copy(x_hbm.at[src_v.at[0, pl.ds(CB, CB)]], rows1, g1)

    def body(jj, carry):
        for b in range(3):
            slot(jj * 3 + b, b)
        return carry

    nfull = (CPT // 3) * 3
    lax.fori_loop(0, CPT // 3, body, 0)
    for j in range(nfull, CPT):
        slot(j, j % 3, static=True)
    pltpu.make_async_copy(rows[(CPT - 2) % 3], acc.at[dst_v.at[0]],
                          ssem[(CPT - 2) % 3]).wait()
    pltpu.make_async_copy(rows[(CPT - 1) % 3], acc.at[dst_v.at[0]],
                          ssem[(CPT - 1) % 3]).wait()
    plsc.subcore_barrier()
    pltpu.sync_copy(acc.at[nsl], out_hbm.at[cid, nsl])


_conv_kernel = functools.partial(
    pl.kernel,
    _conv_body,
    out_type=jax.ShapeDtypeStruct((NC, NP, D), jnp.float32),
    mesh=_sc_mesh,
    compiler_params=pltpu.CompilerParams(needs_layout_passes=False),
    scratch_types=[
        pltpu.VMEM((TPB, 128), jnp.int32),
        pltpu.VMEM((CPT, CB), jnp.int32),
        pltpu.VMEM((TPB, 128), jnp.float32),
        pltpu.VMEM((CB, D), jnp.float32),
        pltpu.VMEM((CB, D), jnp.float32),
        pltpu.VMEM((CB, D), jnp.float32),
        pltpu.VMEM_SHARED((NP, D), jnp.float32),
        pltpu.SemaphoreType.DMA,
        pltpu.SemaphoreType.DMA,
        pltpu.SemaphoreType.DMA,
        pltpu.SemaphoreType.DMA,
        pltpu.SemaphoreType.DMA,
        pltpu.SemaphoreType.DMA,
    ],
)()


@jax.jit
def kernel(rnn_out, edge_index, edge_attr, cent_n_id,
           eW0, eb0, eW1, eb1, node_emb, edge_map_W, edge_map_b,
           gW0, gb0, gW1, gb1, rnn_fc_W, rnn_fc_b, gcn_fc_W, gcn_fc_b):
    f32 = jnp.float32
    rnn = rnn_out[0]
    w = edge_map_W[0]
    wsrc_h, wsrc_id = w[:D], w[D:D + 16]
    wdst_h, wdst_id = w[D + 16:2 * D + 16], w[2 * D + 16:]
    bmap = edge_map_b[0]
    a_src = eW1.T @ wsrc_h
    a_dst = eW1.T @ wdst_h
    c_src = eb1 @ wsrc_h
    c_dst = eb1 @ wdst_h
    q_src = eW0.T @ a_src
    q_dst = eW0.T @ a_dst
    d_src = eb0 @ a_src
    d_dst = eb0 @ a_dst

    cent_p = jnp.pad(cent_n_id, (0, NP - N))
    pad_e = EP - E
    pad_idx = jnp.arange(pad_e, dtype=jnp.int32) % N
    srcb = jnp.concatenate([edge_index[0], pad_idx]).reshape(EB, 128)
    dstb = jnp.concatenate([edge_index[1], pad_idx]).reshape(EB, 128)
    ewb = jnp.concatenate([edge_attr, jnp.zeros((pad_e,), f32)]).reshape(EB, 128)

    qt2 = jnp.stack([q_src, q_dst])
    wet2 = jnp.stack([wsrc_id, wdst_id])
    db = jnp.stack([d_src, d_dst, bmap, jnp.zeros((), f32)])[:, None]
    wyr = jnp.zeros((D, 8), f32).at[:, :7].set(rnn_fc_W.T)
    byr = jnp.zeros((8,), f32).at[:7].set(rnn_fc_b)

    x1, pz, yr8 = _tc_pre(rnn, node_emb, gW0.T, gb0[None], qt2, wet2, db,
                          wyr, byr[None])
    consts = jnp.zeros((16,), f32).at[0].set(c_src).at[1].set(c_dst)

    gateb = _gate_kernel(pz, cent_p, consts, srcb, dstb, ewb)
    dstc = dstb.reshape(EP // CB, CB)
    parts1 = _conv_kernel(x1, srcb, dstc, gateb)
    x2 = _tc_mid(x1, parts1, gW1.T, gb1[None])
    parts2 = _conv_kernel(x2, srcb, dstc, gateb)

    wf8 = jnp.zeros((D, 8), f32).at[:, :7].set(gcn_fc_W.T)
    bf8 = jnp.zeros((8,), f32).at[:7].set(gcn_fc_b)
    y8 = _tc_fin(x2, parts2, yr8, wf8, bf8[None])
    return y8[:N, :7][None]


# speedup vs baseline: 1.1347x; 1.1347x over previous
import functools

import jax
import jax.numpy as jnp
from jax import lax
from jax.experimental import pallas as pl
from jax.experimental.pallas import tpu as pltpu
from jax.experimental.pallas import tpu_sc as plsc

N = 10000
E = 160000
D = 128
NP = 10240
EP = 163840
EB = EP // 128
NC = 2
NS = 16
TPB = EB // (NC * NS)
NPT = NP // NS
PB = EB // NS


BN = 1024
GRID = NP // BN


def _tc_pre_body(rnn_ref, emb_ref, w0_ref, b0_ref, qt_ref, wet_ref, db_ref,
                 wyr_ref, byr_ref, x1_ref, pz_ref, yr_ref):
    r = rnn_ref[...]
    x1_ref[...] = jnp.dot(r, w0_ref[...], preferred_element_type=jnp.float32) + b0_ref[...]
    db = db_ref[...]
    pz_ref[pl.ds(0, 2), :] = lax.dot_general(
        qt_ref[...], r, (((1,), (1,)), ((), ())),
        preferred_element_type=jnp.float32) + db[0:2]
    pz_ref[pl.ds(2, 2), :] = lax.dot_general(
        wet_ref[...], emb_ref[...], (((1,), (1,)), ((), ())),
        preferred_element_type=jnp.float32) + db[2:4]
    yr_ref[...] = jnp.dot(r, wyr_ref[...], preferred_element_type=jnp.float32) + byr_ref[...]


def _tc_pre(rnn, emb, w0t, b0, qt2, wet2, db, wyr, byr):
    return pl.pallas_call(
        _tc_pre_body,
        grid=(GRID,),
        in_specs=[
            pl.BlockSpec((BN, D), lambda i: (i, 0)),
            pl.BlockSpec((BN, 16), lambda i: (i, 0)),
            pl.BlockSpec((D, D), lambda i: (0, 0)),
            pl.BlockSpec((1, D), lambda i: (0, 0)),
            pl.BlockSpec((2, D), lambda i: (0, 0)),
            pl.BlockSpec((2, 16), lambda i: (0, 0)),
            pl.BlockSpec((4, 1), lambda i: (0, 0)),
            pl.BlockSpec((D, 8), lambda i: (0, 0)),
            pl.BlockSpec((1, 8), lambda i: (0, 0)),
        ],
        out_specs=[
            pl.BlockSpec((BN, D), lambda i: (i, 0)),
            pl.BlockSpec((4, BN), lambda i: (0, i)),
            pl.BlockSpec((BN, 8), lambda i: (i, 0)),
        ],
        out_shape=[
            jax.ShapeDtypeStruct((NP, D), jnp.float32),
            jax.ShapeDtypeStruct((4, NP), jnp.float32),
            jax.ShapeDtypeStruct((NP, 8), jnp.float32),
        ],
    )(rnn, emb, w0t, b0, qt2, wet2, db, wyr, byr)


def _tc_mid_body(x1_ref, a0_ref, a1_ref, w_ref, b_ref, x2_ref):
    g = a0_ref[0] + a1_ref[0] - x1_ref[...]
    x2_ref[...] = jnp.dot(g, w_ref[...], preferred_element_type=jnp.float32) + b_ref[...]


def _tc_mid(x1, parts, w1t, b1):
    return pl.pallas_call(
        _tc_mid_body,
        grid=(GRID,),
        in_specs=[
            pl.BlockSpec((BN, D), lambda i: (i, 0)),
            pl.BlockSpec((1, BN, D), lambda i: (0, i, 0)),
            pl.BlockSpec((1, BN, D), lambda i: (1, i, 0)),
            pl.BlockSpec((D, D), lambda i: (0, 0)),
            pl.BlockSpec((1, D), lambda i: (0, 0)),
        ],
        out_specs=pl.BlockSpec((BN, D), lambda i: (i, 0)),
        out_shape=jax.ShapeDtypeStruct((NP, D), jnp.float32),
    )(x1, parts, parts, w1t, b1)


def _tc_fin_body(x2_ref, b0_ref, b1_ref, yr_ref, wf_ref, bf_ref, y_ref):
    g = b0_ref[0] + b1_ref[0] - x2_ref[...]
    y_ref[...] = (yr_ref[...] + bf_ref[...]
                  + jnp.dot(g, wf_ref[...], preferred_element_type=jnp.float32))


def _tc_fin(x2, parts, yr8, wf8, bf8):
    return pl.pallas_call(
        _tc_fin_body,
        grid=(GRID,),
        in_specs=[
            pl.BlockSpec((BN, D), lambda i: (i, 0)),
            pl.BlockSpec((1, BN, D), lambda i: (0, i, 0)),
            pl.BlockSpec((1, BN, D), lambda i: (1, i, 0)),
            pl.BlockSpec((BN, 8), lambda i: (i, 0)),
            pl.BlockSpec((D, 8), lambda i: (0, 0)),
            pl.BlockSpec((1, 8), lambda i: (0, 0)),
        ],
        out_specs=pl.BlockSpec((BN, 8), lambda i: (i, 0)),
        out_shape=jax.ShapeDtypeStruct((NP, 8), jnp.float32),
    )(x2, parts, parts, yr8, wf8, bf8)


_sc_mesh = plsc.VectorSubcoreMesh(core_axis_name="c", subcore_axis_name="s")


def _gate_body(pz, cent, consts, srcb, dstb, ewb, gate_out,
               ps_v, pd_v, zs_v, zd_v, cent_v, src_all, dst_all, ew_all,
               tmp, au_s, au_d, zb, cv, src2, dst2, grow, riota, us_sh, ud_sh):
    cid = lax.axis_index("c")
    sid = lax.axis_index("s")
    pltpu.sync_copy(pz.at[0], ps_v)
    pltpu.sync_copy(pz.at[1], pd_v)
    pltpu.sync_copy(pz.at[2], zs_v)
    pltpu.sync_copy(pz.at[3], zd_v)
    pltpu.sync_copy(cent, cent_v)
    pltpu.sync_copy(consts, cv)
    pb0 = sid * PB
    pltpu.sync_copy(srcb.at[pl.ds(pb0, PB)], src_all)
    pltpu.sync_copy(dstb.at[pl.ds(pb0, PB)], dst_all)
    pltpu.sync_copy(ewb.at[pl.ds(pb0, PB)], ew_all)
    zero16 = jnp.zeros((16,), jnp.float32)
    lane = lax.iota(jnp.int32, 16)
    for r in range(NPT // 128):
        for m in range(8):
            zb[r, pl.ds(m * 16, 16)] = zero16
    for r in range(5):
        riota[pl.ds(r * 16, 16)] = lane + (r * 16)
    shsl = pl.ds(sid * (NPT // 128), NPT // 128)
    pltpu.sync_copy(zb, us_sh.at[shsl])
    pltpu.sync_copy(zb, ud_sh.at[shsl])

    def zero_au():
        def zrow(r, carry):
            for m in range(8):
                sl = pl.ds(m * 16, 16)
                au_s[r, sl] = zero16
                au_d[r, sl] = zero16
            return carry

        lax.fori_loop(0, NP // 128, zrow, 0)

    zero_au()
    plsc.subcore_barrier()

    def edge_pass(j, carry):
        for m in range(8):
            sl = pl.ds(m * 16, 16)
            s16 = src_all[j, sl]
            e16 = ew_all[j, sl]
            d16 = dst_all[j, sl]
            drow = lax.shift_right_logical(d16, 7)
            dcol = lax.bitwise_and(d16, 127)
            plsc.addupdate_scatter(au_s, [drow, dcol],
                                   plsc.load_gather(ps_v, [s16]) * e16)
            plsc.addupdate_scatter(au_d, [drow, dcol],
                                   plsc.load_gather(pd_v, [s16]) * e16)
        return carry

    def reduce_au():
        pltpu.sync_copy(au_s, us_sh.at[riota], add=True)
        pltpu.sync_copy(au_d, ud_sh.at[riota], add=True)

    lax.fori_loop(0, PB, edge_pass, 0)
    reduce_au()
    plsc.subcore_barrier()

    cvec = cv[pl.ds(0, 16)]
    cs = cvec[0]
    cd = cvec[1]
    pltpu.sync_copy(us_sh, tmp)

    def add_c(r, carry):
        for m in range(8):
            sl = pl.ds(m * 16, 16)
            fl = pl.ds(r * 128 + m * 16, 16)
            ps_v[fl] = ps_v[fl] + tmp[r, sl] + cs
        return carry

    lax.fori_loop(0, NP // 128, add_c, 0)
    pltpu.sync_copy(ud_sh, tmp)

    def add_cd(r, carry):
        for m in range(8):
            sl = pl.ds(m * 16, 16)
            fl = pl.ds(r * 128 + m * 16, 16)
            pd_v[fl] = pd_v[fl] + tmp[r, sl] + cd
        return carry

    lax.fori_loop(0, NP // 128, add_cd, 0)
    plsc.subcore_barrier()
    pltpu.sync_copy(zb, us_sh.at[shsl])
    pltpu.sync_copy(zb, ud_sh.at[shsl])
    zero_au()
    plsc.subcore_barrier()

    lax.fori_loop(0, PB, edge_pass, 0)
    reduce_au()
    plsc.subcore_barrier()

    pltpu.sync_copy(us_sh, tmp)

    def add_z(r, carry):
        for m in range(8):
            sl = pl.ds(m * 16, 16)
            fl = pl.ds(r * 128 + m * 16, 16)
            c16 = cent_v[fl]
            ps_v[fl] = ps_v[fl] + tmp[r, sl] + plsc.load_gather(zs_v, [c16])
        return carry

    lax.fori_loop(0, NP // 128, add_z, 0)
    pltpu.sync_copy(ud_sh, tmp)

    def add_zd(r, carry):
        for m in range(8):
            sl = pl.ds(m * 16, 16)
            fl = pl.ds(r * 128 + m * 16, 16)
            c16 = cent_v[fl]
            pd_v[fl] = pd_v[fl] + tmp[r, sl] + plsc.load_gather(zd_v, [c16])
        return carry

    lax.fori_loop(0, NP // 128, add_zd, 0)

    gb0 = cid * (EB // NC) + sid * TPB
    pltpu.sync_copy(srcb.at[pl.ds(gb0, TPB)], src2)
    pltpu.sync_copy(dstb.at[pl.ds(gb0, TPB)], dst2)
    lane = lax.iota(jnp.int32, 16)

    def gate_blk(j, carry):
        base = (gb0 + j) * 128
        for m in range(8):
            sl = pl.ds(m * 16, 16)
            s16 = plsc.load_gather(ps_v, [src2[j, sl]])
            t16 = plsc.load_gather(pd_v, [dst2[j, sl]])
            g = jnp.maximum(s16 + t16, 0.0)
            eidx = lane + (base + m * 16)
            grow[sl] = jnp.where(eidx < E, g, 0.0)
        pltpu.sync_copy(grow, gate_out.at[gb0 + j])
        return carry

    lax.fori_loop(0, TPB, gate_blk, 0)


_gate_kernel = functools.partial(
    pl.kernel,
    _gate_body,
    out_type=jax.ShapeDtypeStruct((EB, 128), jnp.float32),
    mesh=_sc_mesh,
    compiler_params=pltpu.CompilerParams(needs_layout_passes=False),
    scratch_types=[
        pltpu.VMEM((NP,), jnp.float32),
        pltpu.VMEM((NP,), jnp.float32),
        pltpu.VMEM((NP,), jnp.float32),
        pltpu.VMEM((NP,), jnp.float32),
        pltpu.VMEM((NP,), jnp.int32),
        pltpu.VMEM((PB, 128), jnp.int32),
        pltpu.VMEM((PB, 128), jnp.int32),
        pltpu.VMEM((PB, 128), jnp.float32),
        pltpu.VMEM((NP // 128, 128), jnp.float32),
        pltpu.VMEM((NP // 128, 128), jnp.float32),
        pltpu.VMEM((NP // 128, 128), jnp.float32),
        pltpu.VMEM((NPT // 128, 128), jnp.float32),
        pltpu.VMEM((16,), jnp.float32),
        pltpu.VMEM((TPB, 128), jnp.int32),
        pltpu.VMEM((TPB, 128), jnp.int32),
        pltpu.VMEM((128,), jnp.float32),
        pltpu.VMEM((80,), jnp.int32),
        pltpu.VMEM_SHARED((NP // 128, 128), jnp.float32),
        pltpu.VMEM_SHARED((NP // 128, 128), jnp.float32),
    ],
)()


def _conv_body(x_hbm, srcb, dstb, gateb, out_hbm,
               src_v, dst_v, gate_v, rows0, rows1, acc,
               g0, g1, s0, s1):
    cid = lax.axis_index("c")
    sid = lax.axis_index("s")
    wid = cid * NS + sid
    nsl = pl.ds(sid * NPT, NPT)
    pltpu.sync_copy(x_hbm.at[nsl], acc.at[nsl])
    blk0 = wid * TPB
    pltpu.sync_copy(srcb.at[pl.ds(blk0, TPB)], src_v)
    pltpu.sync_copy(dstb.at[pl.ds(blk0, TPB)], dst_v)
    pltpu.sync_copy(gateb.at[pl.ds(blk0, TPB)], gate_v)
    plsc.subcore_barrier()

    rows = [rows0, rows1]
    gsem = [g0, g1]
    ssem = [s0, s1]

    def scale(buf, j):
        def row16(c, c2):
            gvec = gate_v[j, pl.ds(c * 16, 16)]
            for l in range(16):
                g = gvec[l]
                k = c * 16 + l
                for m in range(8):
                    sl = pl.ds(m * 16, 16)
                    buf[k, sl] = buf[k, sl] * g
            return c2

        lax.fori_loop(0, 8, row16, 0)

    pltpu.async_copy(x_hbm.at[src_v.at[0]], rows0, g0)

    def body(jj, carry):
        for b in range(2):
            j = jj * 2 + b
            bn = 1 - b
            pltpu.make_async_copy(x_hbm.at[src_v.at[j]], rows[b], gsem[b]).wait()
            scale(rows[b], j)
            @pl.when(j >= 1)
            def _():
                pltpu.make_async_copy(rows[bn], acc.at[dst_v.at[j]], ssem[bn]).wait()

            @pl.when(j + 1 < TPB)
            def _():
                pltpu.async_copy(x_hbm.at[src_v.at[j + 1]], rows[bn], gsem[bn])
            pltpu.async_copy(rows[b], acc.at[dst_v.at[j]], ssem[b], add=True)
        return carry

    lax.fori_loop(0, TPB // 2, body, 0)
    pltpu.make_async_copy(rows1, acc.at[dst_v.at[0]], s1).wait()
    plsc.subcore_barrier()
    pltpu.sync_copy(acc.at[nsl], out_hbm.at[cid, nsl])


_conv_kernel = functools.partial(
    pl.kernel,
    _conv_body,
    out_type=jax.ShapeDtypeStruct((NC, NP, D), jnp.float32),
    mesh=_sc_mesh,
    compiler_params=pltpu.CompilerParams(needs_layout_passes=False),
    scratch_types=[
        pltpu.VMEM((TPB, 128), jnp.int32),
        pltpu.VMEM((TPB, 128), jnp.int32),
        pltpu.VMEM((TPB, 128), jnp.float32),
        pltpu.VMEM((128, D), jnp.float32),
        pltpu.VMEM((128, D), jnp.float32),
        pltpu.VMEM_SHARED((NP, D), jnp.float32),
        pltpu.SemaphoreType.DMA,
        pltpu.SemaphoreType.DMA,
        pltpu.SemaphoreType.DMA,
        pltpu.SemaphoreType.DMA,
    ],
)()


@jax.jit
def kernel(rnn_out, edge_index, edge_attr, cent_n_id,
           eW0, eb0, eW1, eb1, node_emb, edge_map_W, edge_map_b,
           gW0, gb0, gW1, gb1, rnn_fc_W, rnn_fc_b, gcn_fc_W, gcn_fc_b):
    f32 = jnp.float32
    rnn = rnn_out[0]
    w = edge_map_W[0]
    wsrc_h, wsrc_id = w[:D], w[D:D + 16]
    wdst_h, wdst_id = w[D + 16:2 * D + 16], w[2 * D + 16:]
    bmap = edge_map_b[0]
    a_src = eW1.T @ wsrc_h
    a_dst = eW1.T @ wdst_h
    c_src = eb1 @ wsrc_h
    c_dst = eb1 @ wdst_h
    q_src = eW0.T @ a_src
    q_dst = eW0.T @ a_dst
    d_src = eb0 @ a_src
    d_dst = eb0 @ a_dst

    cent_p = jnp.pad(cent_n_id, (0, NP - N))
    pad_e = EP - E
    pad_idx = jnp.arange(pad_e, dtype=jnp.int32) % N
    srcb = jnp.concatenate([edge_index[0], pad_idx]).reshape(EB, 128)
    dstb = jnp.concatenate([edge_index[1], pad_idx]).reshape(EB, 128)
    ewb = jnp.concatenate([edge_attr, jnp.zeros((pad_e,), f32)]).reshape(EB, 128)

    qt2 = jnp.stack([q_src, q_dst])
    wet2 = jnp.stack([wsrc_id, wdst_id])
    db = jnp.stack([d_src, d_dst, bmap, jnp.zeros((), f32)])[:, None]
    wyr = jnp.zeros((D, 8), f32).at[:, :7].set(rnn_fc_W.T)
    byr = jnp.zeros((8,), f32).at[:7].set(rnn_fc_b)

    x1, pz, yr8 = _tc_pre(rnn, node_emb, gW0.T, gb0[None], qt2, wet2, db,
                          wyr, byr[None])
    consts = jnp.zeros((16,), f32).at[0].set(c_src).at[1].set(c_dst)

    gateb = _gate_kernel(pz, cent_p, consts, srcb, dstb, ewb)
    parts1 = _conv_kernel(x1, srcb, dstb, gateb)
    x2 = _tc_mid(x1, parts1, gW1.T, gb1[None])
    parts2 = _conv_kernel(x2, srcb, dstb, gateb)

    wf8 = jnp.zeros((D, 8), f32).at[:, :7].set(gcn_fc_W.T)
    bf8 = jnp.zeros((8,), f32).at[:7].set(gcn_fc_b)
    y8 = _tc_fin(x2, parts2, yr8, wf8, bf8[None])
    return y8[:N, :7][None]
